# Initial kernel scaffold; baseline (speedup 1.0000x reference)
import jax
import jax.numpy as jnp
from jax.experimental import pallas as pl


def kernel(x, Wg, bg, W1, b1, W2, b2):
    raise NotImplementedError("write your pallas kernel here")


# speedup vs baseline: 1.2703x; 1.2703x over previous
import jax
import jax.numpy as jnp
from jax.experimental import pallas as pl
from jax.experimental.pallas import tpu as pltpu

D_MODEL = 768
NUM_EXPERTS = 8
HIDDEN = 1536
SEQ = 2048
TOKEN_BLOCK = 256
NUM_BLOCKS = SEQ // TOKEN_BLOCK


def _moe_block_kernel(x_ref, wgh_ref, bg_ref, w1_ref, b1_ref,
                      w2_ref, b2_ref, o_ref):
    xb = x_ref[...]
    xh = xb.astype(jnp.bfloat16)
    logits = (
        jnp.dot(xh, wgh_ref[...], preferred_element_type=jnp.float32)
        + bg_ref[...]
    )
    lane = jax.lax.broadcasted_iota(jnp.int32, logits.shape, 1)
    m1 = jnp.max(logits, axis=1, keepdims=True)
    am1 = jnp.min(jnp.where(logits == m1, lane, NUM_EXPERTS), axis=1,
                  keepdims=True)
    masked = jnp.where(lane == am1, -jnp.inf, logits)
    m2 = jnp.max(masked, axis=1, keepdims=True)
    am2 = jnp.min(jnp.where(masked == m2, lane, NUM_EXPERTS), axis=1,
                  keepdims=True)
    p1 = 1.0 / (1.0 + jnp.exp(m2 - m1))
    p2 = 1.0 - p1

    acc = jnp.zeros((TOKEN_BLOCK, D_MODEL), jnp.float32)
    for j in range(NUM_EXPERTS):
        wj = jnp.where(am1 == j, p1, jnp.where(am2 == j, p2, 0.0))
        h = jnp.dot(xh, w1_ref[j], preferred_element_type=jnp.float32)
        h = h + b1_ref[j]
        h = h * jax.nn.sigmoid(h)
        out = jnp.dot(h.astype(jnp.bfloat16), w2_ref[j],
                      preferred_element_type=jnp.float32)
        out = out + b2_ref[j]
        acc = acc + wj * out
    o_ref[...] = acc


def kernel(x, Wg, bg, W1, b1, W2, b2):
    b, s, d = x.shape
    xf = x.reshape(s, d)
    wgh = Wg.astype(jnp.bfloat16)
    w1 = W1.astype(jnp.bfloat16)
    w2 = W2.astype(jnp.bfloat16)
    bg2 = bg.reshape(1, NUM_EXPERTS)
    b1r = b1.reshape(NUM_EXPERTS, 1, HIDDEN)
    b2r = b2.reshape(NUM_EXPERTS, 1, D_MODEL)

    def const3(i):
        return (0, 0, 0)

    def const2(i):
        return (0, 0)

    y = pl.pallas_call(
        _moe_block_kernel,
        grid=(NUM_BLOCKS,),
        in_specs=[
            pl.BlockSpec((TOKEN_BLOCK, D_MODEL), lambda i: (i, 0)),
            pl.BlockSpec((D_MODEL, NUM_EXPERTS), const2),
            pl.BlockSpec((1, NUM_EXPERTS), const2),
            pl.BlockSpec((NUM_EXPERTS, D_MODEL, HIDDEN), const3),
            pl.BlockSpec((NUM_EXPERTS, 1, HIDDEN), const3),
            pl.BlockSpec((NUM_EXPERTS, HIDDEN, D_MODEL), const3),
            pl.BlockSpec((NUM_EXPERTS, 1, D_MODEL), const3),
        ],
        out_specs=pl.BlockSpec((TOKEN_BLOCK, D_MODEL), lambda i: (i, 0)),
        out_shape=jax.ShapeDtypeStruct((s, d), jnp.float32),
        compiler_params=pltpu.CompilerParams(
            dimension_semantics=("parallel",),
        ),
    )(xf, wgh, bg2, w1, b1r, w2, b2r)
    return y.reshape(b, s, d)
